# baseline (speedup 1.0000x reference)
import jax
import jax.numpy as jnp
from jax.experimental import pallas as pl
from jax.experimental.pallas import tpu as pltpu

_BANK = 100000
_BATCH = 16384
_D = 128
_TAIL = _BANK - _BATCH


def _body(feats_hbm, bank_hbm, out_hbm, x_vmem, y_vmem, sem_tail, sem_in, sem_out):
    tail = pltpu.make_async_copy(
        bank_hbm.at[pl.ds(_BATCH, _TAIL)],
        out_hbm.at[pl.ds(_BATCH, _TAIL)],
        sem_tail,
    )
    tail.start()
    feats_in = pltpu.make_async_copy(feats_hbm, x_vmem, sem_in)
    feats_in.start()
    feats_in.wait()
    x = x_vmem[...]
    n2 = jnp.sum(x * x, axis=1, keepdims=True)
    y_vmem[...] = x * jax.lax.rsqrt(jnp.maximum(n2, 1e-24))
    head = pltpu.make_async_copy(y_vmem, out_hbm.at[pl.ds(0, _BATCH)], sem_out)
    head.start()
    head.wait()
    tail.wait()


def kernel(feats, bank):
    return pl.pallas_call(
        _body,
        in_specs=[
            pl.BlockSpec(memory_space=pltpu.MemorySpace.HBM),
            pl.BlockSpec(memory_space=pltpu.MemorySpace.HBM),
        ],
        out_specs=pl.BlockSpec(memory_space=pltpu.MemorySpace.HBM),
        out_shape=jax.ShapeDtypeStruct((_BANK, _D), jnp.float32),
        scratch_shapes=[
            pltpu.VMEM((_BATCH, _D), jnp.float32),
            pltpu.VMEM((_BATCH, _D), jnp.float32),
            pltpu.SemaphoreType.DMA,
            pltpu.SemaphoreType.DMA,
            pltpu.SemaphoreType.DMA,
        ],
    )(feats, bank)
